# baseline (speedup 1.0000x reference)
import functools

import jax
import jax.numpy as jnp
from jax import lax
from jax.experimental import pallas as pl
from jax.experimental.pallas import tpu as pltpu
from jax.experimental.pallas import tpu_sc as plsc

_NUM_LABELS = 3
_D = 1024
_B = 16384
_NC = 2
_NS = 16
_NW = _NC * _NS
_BPW = _B // _NW


def _sc_embedding_lookup(ids, table_flat):
    mesh = plsc.VectorSubcoreMesh(core_axis_name="c", subcore_axis_name="s")

    @functools.partial(
        pl.kernel,
        mesh=mesh,
        out_type=jax.ShapeDtypeStruct((_B * _D,), jnp.float32),
        scratch_types=[
            pltpu.VMEM((_BPW,), jnp.int32),
            pltpu.VMEM((_NUM_LABELS * _D,), jnp.float32),
            pltpu.VMEM((_D,), jnp.float32),
            pltpu.SemaphoreType.DMA,
        ],
    )
    def k(ids_hbm, table_hbm, out_hbm, idx_v, table_v, dummy_v, sem):
        wid = lax.axis_index("s") * _NC + lax.axis_index("c")
        base = wid * _BPW
        pltpu.sync_copy(ids_hbm.at[pl.ds(base, _BPW)], idx_v)
        pltpu.sync_copy(table_hbm, table_v)

        def issue_group(g, carry):
            ids16 = idx_v[pl.ds(g * 16, 16)]
            for j in range(16):
                rid = ids16[j]
                pltpu.async_copy(
                    table_v.at[pl.ds(rid * _D, _D)],
                    out_hbm.at[pl.ds((base + g * 16 + j) * _D, _D)],
                    sem,
                )
            return carry

        lax.fori_loop(0, _BPW // 16, issue_group, 0)

        def drain(r, carry):
            pltpu.make_async_copy(table_hbm.at[pl.ds(0, _D)], dummy_v, sem).wait()
            return carry

        lax.fori_loop(0, _BPW, drain, 0)

    return k(ids, table_flat)


def kernel(sentiment_ids, embedding_table):
    ids = sentiment_ids.astype(jnp.int32)
    table_flat = embedding_table.astype(jnp.float32).reshape(_NUM_LABELS * _D)
    out = _sc_embedding_lookup(ids, table_flat)
    return out.reshape(_B, _D)


# speedup vs baseline: 2.3795x; 2.3795x over previous
import functools

import jax
import jax.numpy as jnp
from jax import lax
from jax.experimental import pallas as pl
from jax.experimental.pallas import tpu as pltpu
from jax.experimental.pallas import tpu_sc as plsc

_NUM_LABELS = 3
_D = 1024
_B = 16384
_NC = 2
_NS = 16
_NW = _NC * _NS
_BPW = _B // _NW


def _sc_embedding_lookup(ids, table_flat):
    mesh = plsc.VectorSubcoreMesh(core_axis_name="c", subcore_axis_name="s")

    @functools.partial(
        pl.kernel,
        mesh=mesh,
        out_type=jax.ShapeDtypeStruct((_B, _D), jnp.float32),
        scratch_types=[
            pltpu.VMEM((_BPW,), jnp.int32),
            pltpu.VMEM((_NUM_LABELS, _D), jnp.float32),
            pltpu.VMEM((1, _D), jnp.float32),
            pltpu.SemaphoreType.DMA,
        ],
    )
    def k(ids_hbm, table_hbm, out_hbm, idx_v, table_v, dummy_v, sem):
        wid = lax.axis_index("s") * _NC + lax.axis_index("c")
        base = wid * _BPW
        pltpu.sync_copy(ids_hbm.at[pl.ds(base, _BPW)], idx_v)
        pltpu.sync_copy(table_hbm, table_v)

        def issue_group(g, carry):
            ids16 = idx_v[pl.ds(g * 16, 16)]
            for j in range(16):
                rid = ids16[j]
                pltpu.async_copy(
                    table_v.at[pl.ds(rid, 1)],
                    out_hbm.at[pl.ds(base + g * 16 + j, 1)],
                    sem,
                )
            return carry

        lax.fori_loop(0, _BPW // 16, issue_group, 0)

        def drain(r, carry):
            pltpu.make_async_copy(table_hbm.at[pl.ds(0, 1)], dummy_v, sem).wait()
            return carry

        lax.fori_loop(0, _BPW, drain, 0)

    return k(ids, table_flat)


def kernel(sentiment_ids, embedding_table):
    ids = sentiment_ids.astype(jnp.int32)
    return _sc_embedding_lookup(ids, embedding_table.astype(jnp.float32))
